# Initial kernel scaffold; baseline (speedup 1.0000x reference)
import jax
import jax.numpy as jnp
from jax.experimental import pallas as pl


def kernel(pos, edge_index, batch, t, params):
    raise NotImplementedError("write your pallas kernel here")


# speedup vs baseline: 1.9117x; 1.9117x over previous
import functools

import numpy as np
import jax
import jax.numpy as jnp
from jax import lax
from jax.experimental import pallas as pl
from jax.experimental.pallas import tpu as pltpu
from jax.experimental.pallas import tpu_sc as plsc

_N = 10000
_E = 320000
_B = 8
_HID = 128
_NG = 50
_NGP = 64
_TDIM = 64
_LAYERS = 2
_CUTOFF = 5.0
_DELTA = _CUTOFF / (_NG - 1)
_COEFF = -0.5 / (_DELTA * _DELTA)

_NPAD = 10240
_NTILE = 256
_ETILE = 1024
_NCORE = 2
_NSUB = 16
_NWORK = _NCORE * _NSUB
_CH = 128
_CPW = 80
_EP = _NWORK * _CH * _CPW
_RPT = _NPAD // _NSUB

_f32 = jnp.float32


def _dot(a, b):
    return jnp.dot(a, b, preferred_element_type=_f32,
                   precision=jax.lax.Precision.HIGHEST)


def _silu(x):
    return x * jax.nn.sigmoid(x)


def _full(shape):
    return pl.BlockSpec(shape, lambda i: (0,) * len(shape))


def _row(w, t=_NTILE):
    return pl.BlockSpec((t, w), lambda i: (i, 0))


def _h0_body(temb, batch, posp, tw1, tb1, tw2, tb2, pw1, pb1, pw2, pb2, out):
    tf = _dot(_silu(_dot(temb[...], tw1[...]) + tb1[...]), tw2[...]) + tb2[...]
    hp = _dot(_silu(_dot(posp[...], pw1[...]) + pb1[...]), pw2[...]) + pb2[...]
    oh = (batch[...] == lax.broadcasted_iota(jnp.int32, (1, _B), 1)).astype(_f32)
    out[...] = hp + _dot(oh, tf)


def _h0_call(temb, batch2, posp, tw1, tb1, tw2, tb2, pw1, pb1, pw2, pb2):
    return pl.pallas_call(
        _h0_body,
        grid=(_NPAD // _NTILE,),
        in_specs=[_full((_B, _TDIM)), _row(1), _row(16),
                  _full((_TDIM, _HID)), _full((1, _HID)), _full((_HID, _HID)), _full((1, _HID)),
                  _full((16, _HID)), _full((1, _HID)), _full((_HID, _HID)), _full((1, _HID))],
        out_specs=_row(_HID),
        out_shape=jax.ShapeDtypeStruct((_NPAD, _HID), _f32),
    )(temb, batch2, posp, tw1, tb1, tw2, tb2, pw1, pb1, pw2, pb2)


def _prep_body(h, wsrc, wdst, eb1, a_out, b_out):
    hh = h[...]
    a_out[...] = _dot(hh, wsrc[...])
    b_out[...] = _dot(hh, wdst[...]) + eb1[...]


def _prep_call(h, wsrc, wdst, eb1):
    return pl.pallas_call(
        _prep_body,
        grid=(_NPAD // _NTILE,),
        in_specs=[_row(_HID), _full((_HID, _HID)), _full((_HID, _HID)), _full((1, _HID))],
        out_specs=[_row(_HID), _row(_HID)],
        out_shape=[jax.ShapeDtypeStruct((_NPAD, _HID), _f32)] * 2,
    )(h, wsrc, wdst, eb1)


def _edge_body(gs, gd, d2, wdf, out):
    dist = jnp.sqrt(d2[...])
    offs = lax.broadcasted_iota(jnp.int32, (1, _NGP), 1).astype(_f32) * _DELTA
    df = jnp.exp(_COEFF * (dist - offs) ** 2)
    z = gs[...] + gd[...] + _dot(df, wdf[...])
    out[...] = _silu(z)


def _edge_call(gs, gd, d2, wdf):
    return pl.pallas_call(
        _edge_body,
        grid=(_EP // _ETILE,),
        in_specs=[_row(_HID, _ETILE), _row(_HID, _ETILE),
                  _row(1, _ETILE), _full((_NGP, _HID))],
        out_specs=_row(_HID, _ETILE),
        out_shape=jax.ShapeDtypeStruct((_EP, _HID), _f32),
    )(gs, gd, d2, wdf)


def _node_body(h, s2, c2, ew2, eb2, nw1h, nw1a, nb1, nw2, nb2, swh, swa, sb, lng, lnb, out):
    s = s2[0, :, :] + s2[1, :, :]
    cnt = c2[0, :, 0:1] + c2[1, :, 0:1]
    agg = _dot(s, ew2[...]) + cnt * eb2[...]
    hh = h[...]
    mid = _silu(_dot(hh, nw1h[...]) + _dot(agg, nw1a[...]) + nb1[...])
    hn = (_dot(mid, nw2[...]) + nb2[...]
          + _dot(hh, swh[...]) + _dot(agg, swa[...]) + sb[...])
    mu = jnp.mean(hn, axis=-1, keepdims=True)
    var = jnp.mean((hn - mu) ** 2, axis=-1, keepdims=True)
    out[...] = (hn - mu) * lax.rsqrt(var + 1e-5) * lng[...] + lnb[...]


def _node_call(h, S, C, ew2, eb2, nw1h, nw1a, nb1, nw2, nb2, swh, swa, sb, lng, lnb):
    return pl.pallas_call(
        _node_body,
        grid=(_NPAD // _NTILE,),
        in_specs=[_row(_HID),
                  pl.BlockSpec((2, _NTILE, _HID), lambda i: (0, i, 0)),
                  pl.BlockSpec((2, _NTILE, _HID), lambda i: (0, i, 0)),
                  _full((_HID, _HID)), _full((1, _HID)),
                  _full((_HID, _HID)), _full((_HID, _HID)), _full((1, _HID)),
                  _full((_HID, _HID)), _full((1, _HID)),
                  _full((_HID, _HID)), _full((_HID, _HID)), _full((1, _HID)),
                  _full((1, _HID)), _full((1, _HID))],
        out_specs=_row(_HID),
        out_shape=jax.ShapeDtypeStruct((_NPAD, _HID), _f32),
    )(h, S, C, ew2, eb2, nw1h, nw1a, nb1, nw2, nb2, swh, swa, sb, lng, lnb)


def _out_body(h, w1, b1, w2, b2, out):
    out[...] = _dot(_silu(_dot(h[...], w1[...]) + b1[...]), w2[...]) + b2[...]


def _out_call(h, w1, b1, w2, b2):
    return pl.pallas_call(
        _out_body,
        grid=(_NPAD // _NTILE,),
        in_specs=[_row(_HID), _full((_HID, _HID)), _full((1, _HID)),
                  _full((_HID, _HID)), _full((1, _HID))],
        out_specs=_row(_HID),
        out_shape=jax.ShapeDtypeStruct((_NPAD, _HID), _f32),
    )(h, w1, b1, w2, b2)


def _sc_mesh():
    return plsc.VectorSubcoreMesh(core_axis_name="c", subcore_axis_name="s",
                                  num_cores=_NCORE, num_subcores=_NSUB)


def _make_gather2(width):
    def body(ta, tb, si, di, oa, ob, si_v, di_v, ra_v, rb_v, sema, semb):
        wid = lax.axis_index("s") * _NCORE + lax.axis_index("c")

        def chunk(ci, carry):
            base = (wid * _CPW + ci) * _CH
            pltpu.sync_copy(si.at[pl.ds(base, _CH)], si_v)
            pltpu.sync_copy(di.at[pl.ds(base, _CH)], di_v)
            ca = pltpu.async_copy(ta.at[si_v], ra_v, sema)
            cb = pltpu.async_copy(tb.at[di_v], rb_v, semb)
            ca.wait()
            cb.wait()
            pltpu.sync_copy(ra_v, oa.at[pl.ds(base, _CH)])
            pltpu.sync_copy(rb_v, ob.at[pl.ds(base, _CH)])
            return carry

        lax.fori_loop(0, _CPW, chunk, 0)

    return pl.kernel(
        body,
        out_type=(jax.ShapeDtypeStruct((_EP, width), _f32),
                  jax.ShapeDtypeStruct((_EP, width), _f32)),
        mesh=_sc_mesh(),
        scratch_types=[
            pltpu.VMEM((_CH,), jnp.int32),
            pltpu.VMEM((_CH,), jnp.int32),
            pltpu.VMEM((_CH, width), _f32),
            pltpu.VMEM((_CH, width), _f32),
            pltpu.SemaphoreType.DMA,
            pltpu.SemaphoreType.DMA,
        ],
    )


_make_gather2 = functools.cache(_make_gather2)


@functools.cache
def _make_edge_geom():
    def body(px, py, pz, si, di, ones, z128, d2_out, c_out,
             px_v, py_v, pz_v, si_v, di_v, d2_v, ones_v, c_sh):
        c = lax.axis_index("c")
        s = lax.axis_index("s")
        wid = s * _NCORE + c
        r0 = s * _RPT
        pltpu.sync_copy(px, px_v)
        pltpu.sync_copy(py, py_v)
        pltpu.sync_copy(pz, pz_v)
        pltpu.sync_copy(ones, ones_v)
        pltpu.sync_copy(z128.at[pl.ds(r0, _RPT)], c_sh.at[pl.ds(r0, _RPT)])
        plsc.subcore_barrier()

        def chunk(ci, carry):
            base = (wid * _CPW + ci) * _CH
            pltpu.sync_copy(si.at[pl.ds(base, _CH)], si_v)
            pltpu.sync_copy(di.at[pl.ds(base, _CH)], di_v)
            for g in range(_CH // 16):
                sidx = si_v[pl.ds(g * 16, 16)]
                didx = di_v[pl.ds(g * 16, 16)]
                ex = plsc.load_gather(px_v, [didx]) - plsc.load_gather(px_v, [sidx])
                ey = plsc.load_gather(py_v, [didx]) - plsc.load_gather(py_v, [sidx])
                ez = plsc.load_gather(pz_v, [didx]) - plsc.load_gather(pz_v, [sidx])
                d2_v[pl.ds(g * 16, 16)] = ex * ex + ey * ey + ez * ez
            pltpu.sync_copy(d2_v, d2_out.at[pl.ds(base, _CH)])
            pltpu.sync_copy(ones_v, c_sh.at[di_v], add=True)
            return carry

        lax.fori_loop(0, _CPW, chunk, 0)
        plsc.subcore_barrier()
        pltpu.sync_copy(c_sh.at[pl.ds(r0, _RPT)], c_out.at[c, pl.ds(r0, _RPT)])

    return pl.kernel(
        body,
        out_type=(jax.ShapeDtypeStruct((_EP,), _f32),
                  jax.ShapeDtypeStruct((_NCORE, _NPAD, _HID), _f32)),
        mesh=_sc_mesh(),
        scratch_types=[
            pltpu.VMEM((_NPAD,), _f32),
            pltpu.VMEM((_NPAD,), _f32),
            pltpu.VMEM((_NPAD,), _f32),
            pltpu.VMEM((_CH,), jnp.int32),
            pltpu.VMEM((_CH,), jnp.int32),
            pltpu.VMEM((_CH,), _f32),
            pltpu.VMEM((_CH, _HID), _f32),
            pltpu.VMEM_SHARED((_NPAD, _HID), _f32),
        ],
        compiler_params=pltpu.CompilerParams(needs_layout_passes=False),
    )


@functools.cache
def _make_scatter():
    def body(u, di, z128, s_out, di_v, u_v, s_sh):
        c = lax.axis_index("c")
        s = lax.axis_index("s")
        r0 = s * _RPT
        pltpu.sync_copy(z128.at[pl.ds(r0, _RPT)], s_sh.at[pl.ds(r0, _RPT)])
        plsc.subcore_barrier()
        half = _EP // _NCORE
        per_tile = half // _NSUB

        def chunk(ci, carry):
            base = c * half + s * per_tile + ci * _CH
            pltpu.sync_copy(di.at[pl.ds(base, _CH)], di_v)
            pltpu.sync_copy(u.at[pl.ds(base, _CH)], u_v)
            pltpu.sync_copy(u_v, s_sh.at[di_v], add=True)
            return carry

        lax.fori_loop(0, per_tile // _CH, chunk, 0)
        plsc.subcore_barrier()
        pltpu.sync_copy(s_sh.at[pl.ds(r0, _RPT)], s_out.at[c, pl.ds(r0, _RPT)])

    return pl.kernel(
        body,
        out_type=jax.ShapeDtypeStruct((_NCORE, _NPAD, _HID), _f32),
        mesh=_sc_mesh(),
        scratch_types=[
            pltpu.VMEM((_CH,), jnp.int32),
            pltpu.VMEM((_CH, _HID), _f32),
            pltpu.VMEM_SHARED((_NPAD, _HID), _f32),
        ],
    )


def kernel(pos, edge_index, batch, t, params):
    p = params
    pos = pos.astype(_f32)
    src = edge_index[0].astype(jnp.int32)
    dst = edge_index[1].astype(jnp.int32)
    pad_e = _EP - _E
    srcp = jnp.concatenate([src, jnp.zeros((pad_e,), jnp.int32)])
    dstp = jnp.concatenate([dst, jnp.full((pad_e,), _N, jnp.int32)])
    posp = jnp.zeros((_NPAD, 16), _f32).at[:_N, :3].set(pos)
    batch2 = jnp.zeros((_NPAD, 1), jnp.int32).at[:_N, 0].set(batch.astype(jnp.int32))

    half = _TDIM // 2
    freqs = jnp.exp(-np.log(10000.0) * jnp.arange(half, dtype=_f32) / half)
    targ = t.astype(_f32)[:, None] * freqs[None, :]
    temb = jnp.concatenate([jnp.sin(targ), jnp.cos(targ)], axis=-1)

    def row1(v):
        return v.reshape(1, -1)

    pw1 = jnp.zeros((16, _HID), _f32).at[:3].set(p['node_w1'])
    h = _h0_call(temb, batch2, posp,
                 p['time_w1'], row1(p['time_b1']), p['time_w2'], row1(p['time_b2']),
                 pw1, row1(p['node_b1']), p['node_w2'], row1(p['node_b2']))

    zeros128 = jnp.zeros((_NPAD, _HID), _f32)
    ones128 = jnp.ones((_CH, _HID), _f32)
    posx = jnp.zeros((_NPAD,), _f32).at[:_N].set(pos[:, 0])
    posy = jnp.zeros((_NPAD,), _f32).at[:_N].set(pos[:, 1])
    posz = jnp.zeros((_NPAD,), _f32).at[:_N].set(pos[:, 2])
    d2, C = _make_edge_geom()(posx, posy, posz, srcp, dstp, ones128, zeros128)
    d2 = d2.reshape(_EP, 1)

    for i in range(_LAYERS):
        lp = p['layers'][i]
        wsrc = lp['edge_w1'][:_HID]
        wdst = lp['edge_w1'][_HID:2 * _HID]
        wdf = jnp.zeros((_NGP, _HID), _f32).at[:_NG].set(lp['edge_w1'][2 * _HID:])
        A, Bt = _prep_call(h, wsrc, wdst, row1(lp['edge_b1']))
        gs, gd = _make_gather2(_HID)(A, Bt, srcp, dstp)
        u = _edge_call(gs, gd, d2, wdf)
        S = _make_scatter()(u, dstp, zeros128)
        h = _node_call(h, S, C, lp['edge_w2'], row1(lp['edge_b2']),
                       lp['node_w1'][:_HID], lp['node_w1'][_HID:], row1(lp['node_b1']),
                       lp['node_w2'], row1(lp['node_b2']),
                       lp['short_w'][:_HID], lp['short_w'][_HID:], row1(lp['short_b']),
                       row1(lp['ln_g']), row1(lp['ln_b']))

    ow2 = jnp.zeros((_HID, _HID), _f32).at[:, :3].set(p['out_w2'])
    ob2 = jnp.zeros((1, _HID), _f32).at[0, :3].set(p['out_b2'])
    velp = _out_call(h, p['out_w1'], row1(p['out_b1']), ow2, ob2)
    return velp[:_N, :3]
